# baseline (speedup 1.0000x reference)
import functools

import jax
import jax.numpy as jnp
from jax import lax
from jax.experimental import pallas as pl
from jax.experimental.pallas import tpu as pltpu
from jax.experimental.pallas import tpu_sc as plsc

SEQ = 200
EMB = 16
BATCH = 4096
FLAT = SEQ * EMB
NROWS = BATCH * SEQ
VOCAB = 1000000
NG = SEQ // 8
QB = BATCH // 16

BK = 32768
BLKS = 31
RSH = (BK // 8).bit_length() - 1
VPAD = BLKS * BK

NUM_CORES = 2
NUM_SUBCORES = 16
NW = NUM_CORES * NUM_SUBCORES

CHUNK = 128
K = 8
G = NROWS // (NW * K * CHUNK)
NCHUNKS = NROWS // CHUNK
CH_PER_W = NCHUNKS // NW


def _repack_body(tt_ref, out_ref):
    x = tt_ref[...]
    y = jnp.concatenate(
        [x[:, (BK // 8) * k:(BK // 8) * (k + 1)] for k in range(8)], axis=0)
    out_ref[...] = jnp.swapaxes(y, 0, 1)


def _repack(table_t):
    return pl.pallas_call(
        _repack_body,
        out_shape=jax.ShapeDtypeStruct((VPAD // 8, 128), jnp.float32),
        grid=(BLKS,),
        in_specs=[pl.BlockSpec((EMB, BK), lambda j: (0, j))],
        out_specs=pl.BlockSpec((BK // 8, 128), lambda j: (j, 0)),
    )(table_t)


def _sc_idx_body(xt_hbm, out_hbm, xloc, idx_v):
    wid = lax.axis_index("s") * NUM_CORES + lax.axis_index("c")
    pltpu.sync_copy(xt_hbm.at[:, pl.ds(wid * 128, 128)], xloc)
    lane = lax.iota(jnp.int32, 16)
    rowpat = lane >> 3
    colpat = lane & 7

    def body(c, carry):
        for j in range(K):
            for v in range(8):
                rvec = 8 * c + colpat
                cvec = (16 * j + 2 * v) + rowpat
                vals = plsc.load_gather(xloc, [rvec, cvec])
                sw = ((vals & ~jnp.int32(BK - 1))
                      | ((vals & (BK // 8 - 1)) << 3)
                      | ((vals >> RSH) & 7))
                idx_v[c, j, pl.ds(16 * v, 16)] = sw
        return carry

    lax.fori_loop(0, NG, body, 0)
    pltpu.sync_copy(idx_v, out_hbm.at[wid])


_sc_idx = pl.kernel(
    _sc_idx_body,
    out_type=jax.ShapeDtypeStruct((NW, NG, K, CHUNK), jnp.int32),
    mesh=plsc.VectorSubcoreMesh(
        core_axis_name="c", subcore_axis_name="s",
        num_cores=NUM_CORES, num_subcores=NUM_SUBCORES),
    scratch_types=[
        pltpu.VMEM((SEQ, 128), jnp.int32),
        pltpu.VMEM((NG, K, CHUNK), jnp.int32),
    ],
    compiler_params=pltpu.CompilerParams(use_tc_tiling_on_sc=False,
                                         needs_layout_passes=False),
)


def _make_sc_gather(c0, nc):
    def body_fn(table_hbm, idx_hbm, out_hbm, idx_v, rows_v,
                gsem0, gsem1, wsem0, wsem1):
        wid = lax.axis_index("s") * NUM_CORES + lax.axis_index("c")
        q0 = wid * K
        gsem = (gsem0, gsem1)
        wsem = (wsem0, wsem1)
        pltpu.sync_copy(idx_hbm.at[wid, pl.ds(c0, nc)], idx_v)

        def fire(c, h):
            for j in range(K):
                pltpu.async_copy(table_hbm.at[idx_v.at[c, j]],
                                 rows_v.at[h, j], gsem[h])

        def wait_gather(h):
            for j in range(K):
                pltpu.make_async_copy(table_hbm.at[idx_v.at[0, j]],
                                      rows_v.at[h, j], gsem[h]).wait()

        def write(c, h):
            pltpu.async_copy(rows_v.at[h], out_hbm.at[c, pl.ds(q0, K)],
                             wsem[h])

        def wait_write(h):
            pltpu.make_async_copy(rows_v.at[h], out_hbm.at[0, pl.ds(q0, K)],
                                  wsem[h]).wait()

        fire(0, 0)
        fire(1, 1)

        def body(p, carry):
            for h in range(2):
                cc = 2 * p + h
                wait_gather(h)
                write(cc, h)

                @pl.when(cc <= nc - 3)
                def _():
                    wait_write(h)
                    fire(cc + 2, h)
            return carry

        lax.fori_loop(0, nc // 2, body, 0)
        if nc % 2:
            wait_gather(0)
            write(nc - 1, 0)
        wait_write((nc - 2) % 2)
        wait_write((nc - 1) % 2)

    return pl.kernel(
        body_fn,
        out_type=jax.ShapeDtypeStruct((nc, QB, CHUNK, EMB), jnp.float32),
        mesh=plsc.VectorSubcoreMesh(
            core_axis_name="c", subcore_axis_name="s",
            num_cores=NUM_CORES, num_subcores=NUM_SUBCORES),
        scratch_types=[
            pltpu.VMEM((nc, K, CHUNK), jnp.int32),
            pltpu.VMEM((2, K, CHUNK, EMB), jnp.float32),
            pltpu.SemaphoreType.DMA,
            pltpu.SemaphoreType.DMA,
            pltpu.SemaphoreType.DMA,
            pltpu.SemaphoreType.DMA,
        ],
        compiler_params=pltpu.CompilerParams(use_tc_tiling_on_sc=False,
                                             needs_layout_passes=False),
    )


NCA = 13
NCB = NG - NCA
_sc_gather_a = _make_sc_gather(0, NCA)
_sc_gather_b = _make_sc_gather(NCA, NCB)


BM = 512


def _tc_matmul_a_body(g_ref, w_ref, out_ref):
    acc = jnp.zeros((BM, 16), jnp.float32)
    for c in range(NCA):
        acc = acc + jnp.dot(g_ref[c], w_ref[128 * c:128 * (c + 1), :],
                            preferred_element_type=jnp.float32)
    out_ref[...] = jnp.swapaxes(acc, 0, 1)


def _tc_matmul_a(g3, W):
    return pl.pallas_call(
        _tc_matmul_a_body,
        out_shape=jax.ShapeDtypeStruct((16, BATCH), jnp.float32),
        grid=(BATCH // BM,),
        in_specs=[
            pl.BlockSpec((NCA, BM, 128), lambda i: (0, i, 0)),
            pl.BlockSpec((FLAT, 16), lambda i: (0, 0)),
        ],
        out_specs=pl.BlockSpec((16, BM), lambda i: (0, i)),
    )(g3, W)


def _tc_matmul_b_body(g_ref, part_ref, pos_ref, w_ref, b_ref, out_ref):
    accp = jnp.dot(pos_ref[...], w_ref[...],
                   preferred_element_type=jnp.float32)
    acc = jnp.zeros((BM, 16), jnp.float32)
    for c in range(NCB):
        acc = acc + jnp.dot(g_ref[c],
                            w_ref[128 * (NCA + c):128 * (NCA + c + 1), :],
                            preferred_element_type=jnp.float32)
    res = acc + accp + b_ref[...]
    out_ref[...] = jnp.swapaxes(res, 0, 1) + part_ref[...]


def _tc_matmul_b(g3, part, pos_flat, W, b2):
    return pl.pallas_call(
        _tc_matmul_b_body,
        out_shape=jax.ShapeDtypeStruct((16, BATCH), jnp.float32),
        grid=(BATCH // BM,),
        in_specs=[
            pl.BlockSpec((NCB, BM, 128), lambda i: (0, i, 0)),
            pl.BlockSpec((16, BM), lambda i: (0, i)),
            pl.BlockSpec((1, FLAT), lambda i: (0, 0)),
            pl.BlockSpec((FLAT, 16), lambda i: (0, 0)),
            pl.BlockSpec((1, 16), lambda i: (0, 0)),
        ],
        out_specs=pl.BlockSpec((16, BM), lambda i: (0, i)),
    )(g3, part, pos_flat, W, b2)


def kernel(x, embed_table, pos_emb, W, b):
    xi = jnp.asarray(x, dtype=jnp.int32)
    idxall = _sc_idx(xi.T)
    packed = _repack(embed_table.T)
    table_lin = packed.reshape(VPAD, EMB)
    ga = _sc_gather_a(table_lin, idxall)
    gb = _sc_gather_b(table_lin, idxall)
    part = _tc_matmul_a(ga.reshape(NCA, BATCH, 128), W)
    out = _tc_matmul_b(gb.reshape(NCB, BATCH, 128), part,
                       pos_emb.reshape(1, FLAT), W, b.reshape(1, 16))
    return out.T


# speedup vs baseline: 1.0284x; 1.0284x over previous
import functools

import jax
import jax.numpy as jnp
from jax import lax
from jax.experimental import pallas as pl
from jax.experimental.pallas import tpu as pltpu
from jax.experimental.pallas import tpu_sc as plsc

SEQ = 200
EMB = 16
BATCH = 4096
FLAT = SEQ * EMB
NROWS = BATCH * SEQ
VOCAB = 1000000
NG = SEQ // 8
QB = BATCH // 16

BK = 32768
BLKS = 31
RSH = (BK // 8).bit_length() - 1
VPAD = BLKS * BK

NUM_CORES = 2
NUM_SUBCORES = 16
NW = NUM_CORES * NUM_SUBCORES

CHUNK = 128
K = 8
G = NROWS // (NW * K * CHUNK)
NCHUNKS = NROWS // CHUNK
CH_PER_W = NCHUNKS // NW


def _repack_body(tt_ref, out_ref):
    x = tt_ref[...]
    y = jnp.concatenate(
        [x[:, (BK // 8) * k:(BK // 8) * (k + 1)] for k in range(8)], axis=0)
    out_ref[...] = jnp.swapaxes(y, 0, 1)


def _repack(table_t):
    return pl.pallas_call(
        _repack_body,
        out_shape=jax.ShapeDtypeStruct((VPAD // 8, 128), jnp.float32),
        grid=(BLKS,),
        in_specs=[pl.BlockSpec((EMB, BK), lambda j: (0, j))],
        out_specs=pl.BlockSpec((BK // 8, 128), lambda j: (j, 0)),
    )(table_t)


def _sc_idx_body(xt_hbm, out_hbm, xloc, idx_v):
    wid = lax.axis_index("s") * NUM_CORES + lax.axis_index("c")
    pltpu.sync_copy(xt_hbm.at[:, pl.ds(wid * 128, 128)], xloc)
    lane = lax.iota(jnp.int32, 16)
    rowpat = lane >> 3
    colpat = lane & 7

    def body(c, carry):
        for j in range(K):
            for v in range(8):
                rvec = 8 * c + colpat
                cvec = (16 * j + 2 * v) + rowpat
                vals = plsc.load_gather(xloc, [rvec, cvec])
                sw = ((vals & ~jnp.int32(BK - 1))
                      | ((vals & (BK // 8 - 1)) << 3)
                      | ((vals >> RSH) & 7))
                idx_v[c, j, pl.ds(16 * v, 16)] = sw
        return carry

    lax.fori_loop(0, NG, body, 0)
    pltpu.sync_copy(idx_v, out_hbm.at[wid])


_sc_idx = pl.kernel(
    _sc_idx_body,
    out_type=jax.ShapeDtypeStruct((NW, NG, K, CHUNK), jnp.int32),
    mesh=plsc.VectorSubcoreMesh(
        core_axis_name="c", subcore_axis_name="s",
        num_cores=NUM_CORES, num_subcores=NUM_SUBCORES),
    scratch_types=[
        pltpu.VMEM((SEQ, 128), jnp.int32),
        pltpu.VMEM((NG, K, CHUNK), jnp.int32),
    ],
    compiler_params=pltpu.CompilerParams(use_tc_tiling_on_sc=False,
                                         needs_layout_passes=False),
)


def _make_sc_gather(c0, nc):
    def body_fn(table_hbm, idx_hbm, out_hbm, idx_v, rows_v,
                gsem0, gsem1, wsem0, wsem1):
        wid = lax.axis_index("s") * NUM_CORES + lax.axis_index("c")
        q0 = wid * K
        gsem = (gsem0, gsem1)
        wsem = (wsem0, wsem1)
        pltpu.sync_copy(idx_hbm.at[wid, pl.ds(c0, nc)], idx_v)

        def fire(c, h):
            for j in range(K):
                pltpu.async_copy(table_hbm.at[idx_v.at[c, j]],
                                 rows_v.at[h, j], gsem[h])

        def wait_gather(h):
            for j in range(K):
                pltpu.make_async_copy(table_hbm.at[idx_v.at[0, j]],
                                      rows_v.at[h, j], gsem[h]).wait()

        def write(c, h):
            pltpu.async_copy(rows_v.at[h], out_hbm.at[c, pl.ds(q0, K)],
                             wsem[h])

        def wait_write(h):
            pltpu.make_async_copy(rows_v.at[h], out_hbm.at[0, pl.ds(q0, K)],
                                  wsem[h]).wait()

        fire(0, 0)
        fire(1, 1)

        def body(p, carry):
            for h in range(2):
                cc = 2 * p + h
                wait_gather(h)
                write(cc, h)

                @pl.when(cc <= nc - 3)
                def _():
                    wait_write(h)
                    fire(cc + 2, h)
            return carry

        lax.fori_loop(0, nc // 2, body, 0)
        if nc % 2:
            wait_gather(0)
            write(nc - 1, 0)
        wait_write((nc - 2) % 2)
        wait_write((nc - 1) % 2)

    return pl.kernel(
        body_fn,
        out_type=jax.ShapeDtypeStruct((nc, QB, CHUNK, EMB), jnp.float32),
        mesh=plsc.VectorSubcoreMesh(
            core_axis_name="c", subcore_axis_name="s",
            num_cores=NUM_CORES, num_subcores=NUM_SUBCORES),
        scratch_types=[
            pltpu.VMEM((nc, K, CHUNK), jnp.int32),
            pltpu.VMEM((2, K, CHUNK, EMB), jnp.float32),
            pltpu.SemaphoreType.DMA,
            pltpu.SemaphoreType.DMA,
            pltpu.SemaphoreType.DMA,
            pltpu.SemaphoreType.DMA,
        ],
        compiler_params=pltpu.CompilerParams(use_tc_tiling_on_sc=False,
                                             needs_layout_passes=False),
    )


_sc_gather = _make_sc_gather(0, NG)


BM = 512


def _tc_matmul_body(g_ref, pos_ref, w_ref, b_ref, out_ref):
    accp = jnp.dot(pos_ref[...], w_ref[...],
                   preferred_element_type=jnp.float32)
    acc = jnp.zeros((BM, 16), jnp.float32)
    for c in range(NG):
        acc = acc + jnp.dot(g_ref[c], w_ref[128 * c:128 * (c + 1), :],
                            preferred_element_type=jnp.float32)
    res = acc + accp + b_ref[...]
    out_ref[...] = jnp.swapaxes(res, 0, 1)


def _tc_matmul(g3, pos_flat, W, b2):
    return pl.pallas_call(
        _tc_matmul_body,
        out_shape=jax.ShapeDtypeStruct((16, BATCH), jnp.float32),
        grid=(BATCH // BM,),
        in_specs=[
            pl.BlockSpec((NG, BM, 128), lambda i: (0, i, 0)),
            pl.BlockSpec((1, FLAT), lambda i: (0, 0)),
            pl.BlockSpec((FLAT, 16), lambda i: (0, 0)),
            pl.BlockSpec((1, 16), lambda i: (0, 0)),
        ],
        out_specs=pl.BlockSpec((16, BM), lambda i: (0, i)),
    )(g3, pos_flat, W, b2)


def kernel(x, embed_table, pos_emb, W, b):
    xi = jnp.asarray(x, dtype=jnp.int32)
    idxall = _sc_idx(xi.T)
    packed = _repack(embed_table.T)
    table_lin = packed.reshape(VPAD, EMB)
    gathered = _sc_gather(table_lin, idxall)
    g3 = gathered.reshape(NG, BATCH, 128)
    out = _tc_matmul(g3, pos_emb.reshape(1, FLAT), W, b.reshape(1, 16))
    return out.T
